# baseline (speedup 1.0000x reference)
import jax
import jax.numpy as jnp
from jax.experimental import pallas as pl
from jax.experimental.pallas import tpu as pltpu

_ROWS = 32768
_COLS = 1024
_BLOCK_ROWS = 1024
_GRID = _ROWS // _BLOCK_ROWS


def _count_kernel(x_ref, y_ref, out_ref):
    out_ref[0, 0, 0] = jnp.sum((x_ref[...] > 0).astype(jnp.int32))
    out_ref[0, 0, 1] = jnp.sum((y_ref[...] > 0).astype(jnp.int32))


def kernel(x, y):
    parts = pl.pallas_call(
        _count_kernel,
        grid=(_GRID,),
        in_specs=[
            pl.BlockSpec((_BLOCK_ROWS, _COLS), lambda i: (i, 0)),
            pl.BlockSpec((_BLOCK_ROWS, _COLS), lambda i: (i, 0)),
        ],
        out_specs=pl.BlockSpec((1, 1, 2), lambda i: (i, 0, 0), memory_space=pltpu.SMEM),
        out_shape=jax.ShapeDtypeStruct((_GRID, 1, 2), jnp.int32),
        compiler_params=pltpu.CompilerParams(
            dimension_semantics=("parallel",),
        ),
    )(x, y)
    totals = parts.sum(axis=(0, 1))
    return jnp.maximum(totals[0], totals[1])
